# baseline (speedup 1.0000x reference)
import jax
import jax.numpy as jnp
from jax import lax
from jax.experimental import pallas as pl
from jax.experimental.pallas import tpu as pltpu
from jax.experimental.pallas import tpu_sc as plsc

N_ROWS = 320000
N_SEG = 10000
D = 128
NC = 2
NS = 16
NW = NC * NS
SEG_PER_W = 320
SEG_PAD = NW * SEG_PER_W
PTR_SLICE = SEG_PER_W + 24
PTR_PAD = (NW - 1) * SEG_PER_W + PTR_SLICE
GROUP = 128
NBUF = 4
LANES = 16
KD = D // LANES
G = SEG_PER_W // LANES
ACC_ROWS = NS * SEG_PER_W + NS


def _pread(ref, i):
    return ref[pl.ds(i, LANES)][0]


def _sc_body(src_hbm, ptr_hbm, out_hbm, ptr_v, marks,
             buf0, buf1, buf2, buf3, ids0, ids1, ids2, ids3, acc,
             sem0, sem1, sem2, sem3, sem_sc):
    sid = lax.axis_index("s")
    cid = lax.axis_index("c")
    wid = sid * NC + cid
    seg0 = pl.multiple_of(wid * SEG_PER_W, 8)
    slot0 = pl.multiple_of(sid * SEG_PER_W, 8)
    dummy = NS * SEG_PER_W + sid

    bufs = (buf0, buf1, buf2, buf3)
    ids_refs = (ids0, ids1, ids2, ids3)
    sems = (sem0, sem1, sem2, sem3)

    pltpu.sync_copy(ptr_hbm.at[pl.ds(seg0, PTR_SLICE)], ptr_v)
    row_lo = _pread(ptr_v, 0)
    row_hi = _pread(ptr_v, SEG_PER_W)

    zf = jnp.zeros((LANES,), jnp.float32)
    zi = jnp.zeros((LANES,), jnp.int32)
    ones = jnp.ones((LANES,), jnp.int32)
    iota = lax.iota(jnp.int32, LANES)

    def zero_body(s, _):
        for k in range(KD):
            buf0[s, pl.ds(k * LANES, LANES)] = zf
        return 0

    lax.fori_loop(0, GROUP, zero_body, 0)
    for p, m in ((0, 128), (128, 128), (256, 64)):
        pltpu.sync_copy(buf0.at[pl.ds(0, m)], acc.at[pl.ds(slot0 + p, m)])

    row_lo_a = (row_lo // 8) * 8
    ngrp = (row_hi - row_lo_a + GROUP - 1) // GROUP

    def grp_off(t):
        off = row_lo_a + t * GROUP
        return off, pl.multiple_of(jnp.minimum(off, N_ROWS - GROUP), 8)

    def start_load(t, buf, sem):
        _, off_c = grp_off(t)
        pltpu.async_copy(src_hbm.at[pl.ds(off_c, GROUP)], buf, sem)

    @pl.when(0 < ngrp)
    def _():
        start_load(0, buf0, sem0)

    @pl.when(1 < ngrp)
    def _():
        start_load(1, buf1, sem1)

    def ring_body(g, base):
        for k in range(NBUF):
            t = g * NBUF + k
            kk = (k + 1) % NBUF

            @pl.when((t >= 3) & (t - 3 < ngrp))
            def _(kk=kk):
                pltpu.make_async_copy(
                    bufs[kk], acc.at[ids_refs[kk]], sem_sc
                ).wait()

            @pl.when(t + 1 < ngrp)
            def _(kk=kk, t=t):
                start_load(t + 1, bufs[kk], sems[kk])

            def fire(bs, t=t, k=k):
                off, off_c = grp_off(t)
                for j in range(GROUP // LANES):
                    marks[pl.ds(j * LANES, LANES)] = zi
                hi = off_c + GROUP

                def sm(q, _):
                    starts = ptr_v[pl.ds(q * LANES, LANES)]
                    m = (starts >= off) & (starts < hi)
                    plsc.addupdate_scatter(marks, [starts - off_c], ones, mask=m)
                    return 0

                lax.fori_loop(0, G, sm, 0)

                for j in range(GROUP // LANES):
                    mk = marks[pl.ds(j * LANES, LANES)]
                    csum = plsc.cumsum(mk)
                    idx16 = off_c + j * LANES + iota
                    valid = (idx16 >= off) & (idx16 >= row_lo) & (idx16 < row_hi)
                    slot = jnp.where(valid, slot0 + bs + csum - 1, dummy)
                    ids_refs[k][pl.ds(j * LANES, LANES)] = slot
                    bs = bs + csum[15]

                pltpu.make_async_copy(
                    src_hbm.at[pl.ds(off_c, GROUP)], bufs[k], sems[k]
                ).wait()
                pltpu.async_copy(
                    bufs[k], acc.at[ids_refs[k]], sem_sc, add=True
                )
                return bs

            base = lax.cond(t < ngrp, fire, lambda bs: bs, base)
        return base

    lax.fori_loop(0, (ngrp + 3 + NBUF - 1) // NBUF, ring_body, 0)

    for p, m in ((0, 128), (128, 128), (256, 64)):
        pltpu.sync_copy(acc.at[pl.ds(slot0 + p, m)], buf0.at[pl.ds(0, m)])

        def div_body(g2, _, p=p):
            cur16 = ptr_v[pl.ds(p + g2 * LANES, LANES)]
            nxt16 = plsc.load_gather(ptr_v, [p + g2 * LANES + 1 + iota])
            cntf = (nxt16 - cur16).astype(jnp.float32)
            recip = 1.0 / jnp.maximum(cntf, 1.0)
            for jj in range(LANES):
                rv = jnp.full((LANES,), recip[jj], jnp.float32)
                for k in range(KD):
                    sl = pl.ds(k * LANES, LANES)
                    buf0[g2 * LANES + jj, sl] = buf0[g2 * LANES + jj, sl] * rv
            return 0

        lax.fori_loop(0, m // LANES, div_body, 0)
        pltpu.sync_copy(buf0.at[pl.ds(0, m)], out_hbm.at[pl.ds(seg0 + p, m)])


@jax.jit
def _run(src, ptr_pad):
    mesh = plsc.VectorSubcoreMesh(core_axis_name="c", subcore_axis_name="s")
    k = pl.kernel(
        _sc_body,
        out_type=jax.ShapeDtypeStruct((SEG_PAD, D), jnp.float32),
        mesh=mesh,
        scratch_types=[
            pltpu.VMEM((PTR_SLICE,), jnp.int32),
            pltpu.VMEM((GROUP,), jnp.int32),
            pltpu.VMEM((GROUP, D), jnp.float32),
            pltpu.VMEM((GROUP, D), jnp.float32),
            pltpu.VMEM((GROUP, D), jnp.float32),
            pltpu.VMEM((GROUP, D), jnp.float32),
            pltpu.VMEM((GROUP,), jnp.int32),
            pltpu.VMEM((GROUP,), jnp.int32),
            pltpu.VMEM((GROUP,), jnp.int32),
            pltpu.VMEM((GROUP,), jnp.int32),
            pltpu.VMEM_SHARED((ACC_ROWS, D), jnp.float32),
            pltpu.SemaphoreType.DMA,
            pltpu.SemaphoreType.DMA,
            pltpu.SemaphoreType.DMA,
            pltpu.SemaphoreType.DMA,
            pltpu.SemaphoreType.DMA,
        ],
        compiler_params=pltpu.CompilerParams(needs_layout_passes=False),
    )
    return k(src, ptr_pad)


def kernel(src, indptr):
    ptr = indptr.astype(jnp.int32)
    ptr_pad = jnp.concatenate(
        [ptr, jnp.full((PTR_PAD - ptr.shape[0],), ptr[-1], jnp.int32)]
    )
    out = _run(src, ptr_pad)
    return out[:N_SEG]


# speedup vs baseline: 1.0671x; 1.0671x over previous
import jax
import jax.numpy as jnp
from jax import lax
from jax.experimental import pallas as pl
from jax.experimental.pallas import tpu as pltpu
from jax.experimental.pallas import tpu_sc as plsc

N_ROWS = 320000
N_SEG = 10000
D = 128
NC = 2
NS = 16
NW = NC * NS
SEG_PER_W = 320
SEG_PAD = NW * SEG_PER_W
PTR_SLICE = SEG_PER_W + 24
PTR_PAD = (NW - 1) * SEG_PER_W + PTR_SLICE
GROUP = 128
NBUF = 5
LANES = 16
KD = D // LANES
G = SEG_PER_W // LANES
ACC_ROWS = NS * SEG_PER_W + NS


def _pread(ref, i):
    return ref[pl.ds(i, LANES)][0]


def _sc_body(src_hbm, ptr_hbm, out_hbm, ptr_v, marks,
             buf0, buf1, buf2, buf3, buf4, ids0, ids1, ids2, ids3, ids4, acc,
             sem0, sem1, sem2, sem3, sem4, sem_sc):
    sid = lax.axis_index("s")
    cid = lax.axis_index("c")
    wid = sid * NC + cid
    seg0 = pl.multiple_of(wid * SEG_PER_W, 8)
    slot0 = pl.multiple_of(sid * SEG_PER_W, 8)
    dummy = NS * SEG_PER_W + sid

    bufs = (buf0, buf1, buf2, buf3, buf4)
    ids_refs = (ids0, ids1, ids2, ids3, ids4)
    sems = (sem0, sem1, sem2, sem3, sem4)

    pltpu.sync_copy(ptr_hbm.at[pl.ds(seg0, PTR_SLICE)], ptr_v)
    row_lo = _pread(ptr_v, 0)
    row_hi = _pread(ptr_v, SEG_PER_W)

    zf = jnp.zeros((LANES,), jnp.float32)
    zi = jnp.zeros((LANES,), jnp.int32)
    ones = jnp.ones((LANES,), jnp.int32)
    iota = lax.iota(jnp.int32, LANES)

    def zero_body(s, _):
        for k in range(KD):
            buf0[s, pl.ds(k * LANES, LANES)] = zf
        return 0

    lax.fori_loop(0, GROUP, zero_body, 0)
    for p, m in ((0, 128), (128, 128), (256, 64)):
        pltpu.sync_copy(buf0.at[pl.ds(0, m)], acc.at[pl.ds(slot0 + p, m)])

    row_lo_a = (row_lo // 8) * 8
    ngrp = (row_hi - row_lo_a + GROUP - 1) // GROUP

    def grp_off(t):
        off = row_lo_a + t * GROUP
        return off, pl.multiple_of(jnp.minimum(off, N_ROWS - GROUP), 8)

    def start_load(t, buf, sem):
        _, off_c = grp_off(t)
        pltpu.async_copy(src_hbm.at[pl.ds(off_c, GROUP)], buf, sem)

    @pl.when(0 < ngrp)
    def _():
        start_load(0, buf0, sem0)

    @pl.when(1 < ngrp)
    def _():
        start_load(1, buf1, sem1)

    @pl.when(2 < ngrp)
    def _():
        start_load(2, buf2, sem2)

    def ring_body(g, base):
        for k in range(NBUF):
            t = g * NBUF + k
            kk = (k + 3) % NBUF

            @pl.when((t >= 2) & (t - 2 < ngrp))
            def _(kk=kk):
                pltpu.make_async_copy(
                    bufs[kk], acc.at[ids_refs[kk]], sem_sc
                ).wait()

            @pl.when(t + 3 < ngrp)
            def _(kk=kk, t=t):
                start_load(t + 3, bufs[kk], sems[kk])

            def fire(bs, t=t, k=k):
                off, off_c = grp_off(t)
                for j in range(GROUP // LANES):
                    marks[pl.ds(j * LANES, LANES)] = zi
                hi = off_c + GROUP

                def sm(q, _):
                    starts = ptr_v[pl.ds(q * LANES, LANES)]
                    m = (starts >= off) & (starts < hi)
                    plsc.addupdate_scatter(marks, [starts - off_c], ones, mask=m)
                    return 0

                lax.fori_loop(0, G, sm, 0)

                for j in range(GROUP // LANES):
                    mk = marks[pl.ds(j * LANES, LANES)]
                    csum = plsc.cumsum(mk)
                    idx16 = off_c + j * LANES + iota
                    valid = (idx16 >= off) & (idx16 >= row_lo) & (idx16 < row_hi)
                    slot = jnp.where(valid, slot0 + bs + csum - 1, dummy)
                    ids_refs[k][pl.ds(j * LANES, LANES)] = slot
                    bs = bs + csum[15]

                pltpu.make_async_copy(
                    src_hbm.at[pl.ds(off_c, GROUP)], bufs[k], sems[k]
                ).wait()
                pltpu.async_copy(
                    bufs[k], acc.at[ids_refs[k]], sem_sc, add=True
                )
                return bs

            base = lax.cond(t < ngrp, fire, lambda bs: bs, base)
        return base

    lax.fori_loop(0, (ngrp + 2 + NBUF - 1) // NBUF, ring_body, 0)

    for p, m in ((0, 128), (128, 128), (256, 64)):
        pltpu.sync_copy(acc.at[pl.ds(slot0 + p, m)], buf0.at[pl.ds(0, m)])

        def div_body(g2, _, p=p):
            cur16 = ptr_v[pl.ds(p + g2 * LANES, LANES)]
            nxt16 = plsc.load_gather(ptr_v, [p + g2 * LANES + 1 + iota])
            cntf = (nxt16 - cur16).astype(jnp.float32)
            recip = 1.0 / jnp.maximum(cntf, 1.0)
            for jj in range(LANES):
                rv = jnp.full((LANES,), recip[jj], jnp.float32)
                for k in range(KD):
                    sl = pl.ds(k * LANES, LANES)
                    buf0[g2 * LANES + jj, sl] = buf0[g2 * LANES + jj, sl] * rv
            return 0

        lax.fori_loop(0, m // LANES, div_body, 0)
        pltpu.sync_copy(buf0.at[pl.ds(0, m)], out_hbm.at[pl.ds(seg0 + p, m)])


@jax.jit
def _run(src, ptr_pad):
    mesh = plsc.VectorSubcoreMesh(core_axis_name="c", subcore_axis_name="s")
    k = pl.kernel(
        _sc_body,
        out_type=jax.ShapeDtypeStruct((SEG_PAD, D), jnp.float32),
        mesh=mesh,
        scratch_types=[
            pltpu.VMEM((PTR_SLICE,), jnp.int32),
            pltpu.VMEM((GROUP,), jnp.int32),
            pltpu.VMEM((GROUP, D), jnp.float32),
            pltpu.VMEM((GROUP, D), jnp.float32),
            pltpu.VMEM((GROUP, D), jnp.float32),
            pltpu.VMEM((GROUP, D), jnp.float32),
            pltpu.VMEM((GROUP, D), jnp.float32),
            pltpu.VMEM((GROUP,), jnp.int32),
            pltpu.VMEM((GROUP,), jnp.int32),
            pltpu.VMEM((GROUP,), jnp.int32),
            pltpu.VMEM((GROUP,), jnp.int32),
            pltpu.VMEM((GROUP,), jnp.int32),
            pltpu.VMEM_SHARED((ACC_ROWS, D), jnp.float32),
            pltpu.SemaphoreType.DMA,
            pltpu.SemaphoreType.DMA,
            pltpu.SemaphoreType.DMA,
            pltpu.SemaphoreType.DMA,
            pltpu.SemaphoreType.DMA,
            pltpu.SemaphoreType.DMA,
        ],
        compiler_params=pltpu.CompilerParams(needs_layout_passes=False),
    )
    return k(src, ptr_pad)


def kernel(src, indptr):
    ptr = indptr.astype(jnp.int32)
    ptr_pad = jnp.concatenate(
        [ptr, jnp.full((PTR_PAD - ptr.shape[0],), ptr[-1], jnp.int32)]
    )
    out = _run(src, ptr_pad)
    return out[:N_SEG]


# speedup vs baseline: 1.0706x; 1.0032x over previous
import jax
import jax.numpy as jnp
from jax import lax
from jax.experimental import pallas as pl
from jax.experimental.pallas import tpu as pltpu
from jax.experimental.pallas import tpu_sc as plsc

N_ROWS = 320000
N_SEG = 10000
D = 128
NC = 2
NS = 16
NW = NC * NS
SEG_PER_W = 320
SEG_PAD = NW * SEG_PER_W
PTR_SLICE = SEG_PER_W + 24
PTR_PAD = (NW - 1) * SEG_PER_W + PTR_SLICE
GROUP = 128
NBUF = 4
LANES = 16
KD = D // LANES
G = SEG_PER_W // LANES
ACC_ROWS = NS * SEG_PER_W + NS


def _pread(ref, i):
    return ref[pl.ds(i, LANES)][0]


def _sc_body(src_hbm, ptr_hbm, out_hbm, ptr_v, marks,
             buf0, buf1, buf2, buf3, ids0, ids1, ids2, ids3, acc,
             sem0, sem1, sem2, sem3, sem_sc):
    sid = lax.axis_index("s")
    cid = lax.axis_index("c")
    wid = sid * NC + cid
    seg0 = pl.multiple_of(wid * SEG_PER_W, 8)
    slot0 = pl.multiple_of(sid * SEG_PER_W, 8)
    dummy = NS * SEG_PER_W + sid

    bufs = (buf0, buf1, buf2, buf3)
    ids_refs = (ids0, ids1, ids2, ids3)
    sems = (sem0, sem1, sem2, sem3)

    pltpu.sync_copy(ptr_hbm.at[pl.ds(seg0, PTR_SLICE)], ptr_v)
    row_lo = _pread(ptr_v, 0)
    row_hi = _pread(ptr_v, SEG_PER_W)

    zf = jnp.zeros((LANES,), jnp.float32)
    zi = jnp.zeros((LANES,), jnp.int32)
    ones = jnp.ones((LANES,), jnp.int32)
    iota = lax.iota(jnp.int32, LANES)

    def zero_body(s, _):
        for k in range(KD):
            buf0[s, pl.ds(k * LANES, LANES)] = zf
        return 0

    lax.fori_loop(0, GROUP, zero_body, 0)
    for p, m in ((0, 128), (128, 128), (256, 64)):
        pltpu.sync_copy(buf0.at[pl.ds(0, m)], acc.at[pl.ds(slot0 + p, m)])

    row_lo_a = (row_lo // 8) * 8
    ngrp = (row_hi - row_lo_a + GROUP - 1) // GROUP

    def grp_off(t):
        off = row_lo_a + t * GROUP
        return off, pl.multiple_of(jnp.minimum(off, N_ROWS - GROUP), 8)

    def start_load(t, buf, sem):
        _, off_c = grp_off(t)
        pltpu.async_copy(src_hbm.at[pl.ds(off_c, GROUP)], buf, sem)

    @pl.when(0 < ngrp)
    def _():
        start_load(0, buf0, sem0)

    @pl.when(1 < ngrp)
    def _():
        start_load(1, buf1, sem1)

    def ring_body(g, base):
        for k in range(NBUF):
            t = g * NBUF + k
            kk = (k + 2) % NBUF

            @pl.when((t >= 2) & (t - 2 < ngrp))
            def _(kk=kk):
                pltpu.make_async_copy(
                    bufs[kk], acc.at[ids_refs[kk]], sem_sc
                ).wait()

            @pl.when(t + 2 < ngrp)
            def _(kk=kk, t=t):
                start_load(t + 2, bufs[kk], sems[kk])

            def fire(bs, t=t, k=k):
                off, off_c = grp_off(t)
                for j in range(GROUP // LANES):
                    marks[pl.ds(j * LANES, LANES)] = zi
                hi = off_c + GROUP

                def sm(q, _):
                    starts = ptr_v[pl.ds(q * LANES, LANES)]
                    m = (starts >= off) & (starts < hi)
                    plsc.addupdate_scatter(marks, [starts - off_c], ones, mask=m)
                    return 0

                lax.fori_loop(0, G, sm, 0)

                for j in range(GROUP // LANES):
                    mk = marks[pl.ds(j * LANES, LANES)]
                    csum = plsc.cumsum(mk)
                    idx16 = off_c + j * LANES + iota
                    valid = (idx16 >= off) & (idx16 >= row_lo) & (idx16 < row_hi)
                    slot = jnp.where(valid, slot0 + bs + csum - 1, dummy)
                    ids_refs[k][pl.ds(j * LANES, LANES)] = slot
                    bs = bs + csum[15]

                pltpu.make_async_copy(
                    src_hbm.at[pl.ds(off_c, GROUP)], bufs[k], sems[k]
                ).wait()
                pltpu.async_copy(
                    bufs[k], acc.at[ids_refs[k]], sem_sc, add=True
                )
                return bs

            base = lax.cond(t < ngrp, fire, lambda bs: bs, base)
        return base

    lax.fori_loop(0, (ngrp + 2 + NBUF - 1) // NBUF, ring_body, 0)

    for p, m in ((0, 128), (128, 128), (256, 64)):
        pltpu.sync_copy(acc.at[pl.ds(slot0 + p, m)], buf0.at[pl.ds(0, m)])

        def div_body(g2, _, p=p):
            cur16 = ptr_v[pl.ds(p + g2 * LANES, LANES)]
            nxt16 = plsc.load_gather(ptr_v, [p + g2 * LANES + 1 + iota])
            cntf = (nxt16 - cur16).astype(jnp.float32)
            recip = 1.0 / jnp.maximum(cntf, 1.0)
            for jj in range(LANES):
                rv = jnp.full((LANES,), recip[jj], jnp.float32)
                for k in range(KD):
                    sl = pl.ds(k * LANES, LANES)
                    buf0[g2 * LANES + jj, sl] = buf0[g2 * LANES + jj, sl] * rv
            return 0

        lax.fori_loop(0, m // LANES, div_body, 0)
        pltpu.sync_copy(buf0.at[pl.ds(0, m)], out_hbm.at[pl.ds(seg0 + p, m)])


@jax.jit
def _run(src, ptr_pad):
    mesh = plsc.VectorSubcoreMesh(core_axis_name="c", subcore_axis_name="s")
    k = pl.kernel(
        _sc_body,
        out_type=jax.ShapeDtypeStruct((SEG_PAD, D), jnp.float32),
        mesh=mesh,
        scratch_types=[
            pltpu.VMEM((PTR_SLICE,), jnp.int32),
            pltpu.VMEM((GROUP,), jnp.int32),
            pltpu.VMEM((GROUP, D), jnp.float32),
            pltpu.VMEM((GROUP, D), jnp.float32),
            pltpu.VMEM((GROUP, D), jnp.float32),
            pltpu.VMEM((GROUP, D), jnp.float32),
            pltpu.VMEM((GROUP,), jnp.int32),
            pltpu.VMEM((GROUP,), jnp.int32),
            pltpu.VMEM((GROUP,), jnp.int32),
            pltpu.VMEM((GROUP,), jnp.int32),
            pltpu.VMEM_SHARED((ACC_ROWS, D), jnp.float32),
            pltpu.SemaphoreType.DMA,
            pltpu.SemaphoreType.DMA,
            pltpu.SemaphoreType.DMA,
            pltpu.SemaphoreType.DMA,
            pltpu.SemaphoreType.DMA,
        ],
        compiler_params=pltpu.CompilerParams(needs_layout_passes=False),
    )
    return k(src, ptr_pad)


def kernel(src, indptr):
    ptr = indptr.astype(jnp.int32)
    ptr_pad = jnp.concatenate(
        [ptr, jnp.full((PTR_PAD - ptr.shape[0],), ptr[-1], jnp.int32)]
    )
    out = _run(src, ptr_pad)
    return out[:N_SEG]


# speedup vs baseline: 1.0711x; 1.0005x over previous
import jax
import jax.numpy as jnp
from jax import lax
from jax.experimental import pallas as pl
from jax.experimental.pallas import tpu as pltpu
from jax.experimental.pallas import tpu_sc as plsc

N_ROWS = 320000
N_SEG = 10000
D = 128
NC = 2
NS = 16
NW = NC * NS
SEG_PER_W = 320
SEG_PAD = NW * SEG_PER_W
PTR_SLICE = SEG_PER_W + 24
PTR_PAD = (NW - 1) * SEG_PER_W + PTR_SLICE
GROUP = 128
NBUF = 4
LANES = 16
KD = D // LANES
G = SEG_PER_W // LANES
ACC_ROWS = NS * SEG_PER_W + NS


def _pread(ref, i):
    return ref[pl.ds(i, LANES)][0]


def _sc_body(src_hbm, ptr_hbm, out_hbm, ptr_v, marks,
             buf0, buf1, buf2, buf3, ids0, ids1, ids2, ids3, acc,
             sem0, sem1, sem2, sem3, sem_sc):
    sid = lax.axis_index("s")
    cid = lax.axis_index("c")
    wid = sid * NC + cid
    seg0 = pl.multiple_of(wid * SEG_PER_W, 8)
    slot0 = pl.multiple_of(sid * SEG_PER_W, 8)
    dummy = NS * SEG_PER_W + sid

    bufs = (buf0, buf1, buf2, buf3)
    ids_refs = (ids0, ids1, ids2, ids3)
    sems = (sem0, sem1, sem2, sem3)

    pltpu.sync_copy(ptr_hbm.at[pl.ds(seg0, PTR_SLICE)], ptr_v)
    row_lo = _pread(ptr_v, 0)
    row_hi = _pread(ptr_v, SEG_PER_W)

    zf = jnp.zeros((LANES,), jnp.float32)
    zi = jnp.zeros((LANES,), jnp.int32)
    ones = jnp.ones((LANES,), jnp.int32)
    iota = lax.iota(jnp.int32, LANES)

    def zero_body(s, _):
        for k in range(KD):
            buf0[s, pl.ds(k * LANES, LANES)] = zf
        return 0

    lax.fori_loop(0, GROUP, zero_body, 0)
    for p, m in ((0, 128), (128, 128), (256, 64)):
        pltpu.sync_copy(buf0.at[pl.ds(0, m)], acc.at[pl.ds(slot0 + p, m)])

    row_lo_a = (row_lo // 8) * 8
    ngrp = (row_hi - row_lo_a + GROUP - 1) // GROUP

    def grp_off(t):
        off = row_lo_a + t * GROUP
        return off, pl.multiple_of(jnp.minimum(off, N_ROWS - GROUP), 8)

    def start_load(t, buf, sem):
        _, off_c = grp_off(t)
        pltpu.async_copy(src_hbm.at[pl.ds(off_c, GROUP)], buf, sem)

    @pl.when(0 < ngrp)
    def _():
        start_load(0, buf0, sem0)

    @pl.when(1 < ngrp)
    def _():
        start_load(1, buf1, sem1)

    def ring_body(g, base):
        for k in range(NBUF):
            t = g * NBUF + k
            kk = (k + 2) % NBUF

            @pl.when((t >= 2) & (t - 2 < ngrp))
            def _(kk=kk):
                pltpu.make_async_copy(
                    bufs[kk], acc.at[ids_refs[kk]], sem_sc
                ).wait()

            @pl.when(t + 2 < ngrp)
            def _(kk=kk, t=t):
                start_load(t + 2, bufs[kk], sems[kk])

            def fire(bs, t=t, k=k):
                off, off_c = grp_off(t)
                for j in range(GROUP // LANES):
                    marks[pl.ds(j * LANES, LANES)] = zi
                hi = off_c + GROUP

                def sm(q, _):
                    starts = ptr_v[pl.ds(q * LANES, LANES)]
                    m = (starts >= off) & (starts < hi)
                    plsc.addupdate_scatter(marks, [starts - off_c], ones, mask=m)
                    return 0

                lax.fori_loop(0, G, sm, 0)

                for j in range(GROUP // LANES):
                    mk = marks[pl.ds(j * LANES, LANES)]
                    csum = plsc.cumsum(mk)
                    idx16 = off_c + j * LANES + iota
                    valid = (idx16 >= off) & (idx16 >= row_lo) & (idx16 < row_hi)
                    slot = jnp.where(valid, slot0 + bs + csum - 1, dummy)
                    ids_refs[k][pl.ds(j * LANES, LANES)] = slot
                    bs = bs + csum[15]

                pltpu.make_async_copy(
                    src_hbm.at[pl.ds(off_c, GROUP)], bufs[k], sems[k]
                ).wait()
                pltpu.async_copy(
                    bufs[k], acc.at[ids_refs[k]], sem_sc, priority=1, add=True
                )
                return bs

            base = lax.cond(t < ngrp, fire, lambda bs: bs, base)
        return base

    lax.fori_loop(0, (ngrp + 2 + NBUF - 1) // NBUF, ring_body, 0)

    for p, m in ((0, 128), (128, 128), (256, 64)):
        pltpu.sync_copy(acc.at[pl.ds(slot0 + p, m)], buf0.at[pl.ds(0, m)])

        def div_body(g2, _, p=p):
            cur16 = ptr_v[pl.ds(p + g2 * LANES, LANES)]
            nxt16 = plsc.load_gather(ptr_v, [p + g2 * LANES + 1 + iota])
            cntf = (nxt16 - cur16).astype(jnp.float32)
            recip = 1.0 / jnp.maximum(cntf, 1.0)
            for jj in range(LANES):
                rv = jnp.full((LANES,), recip[jj], jnp.float32)
                for k in range(KD):
                    sl = pl.ds(k * LANES, LANES)
                    buf0[g2 * LANES + jj, sl] = buf0[g2 * LANES + jj, sl] * rv
            return 0

        lax.fori_loop(0, m // LANES, div_body, 0)
        pltpu.sync_copy(buf0.at[pl.ds(0, m)], out_hbm.at[pl.ds(seg0 + p, m)])


@jax.jit
def _run(src, ptr_pad):
    mesh = plsc.VectorSubcoreMesh(core_axis_name="c", subcore_axis_name="s")
    k = pl.kernel(
        _sc_body,
        out_type=jax.ShapeDtypeStruct((SEG_PAD, D), jnp.float32),
        mesh=mesh,
        scratch_types=[
            pltpu.VMEM((PTR_SLICE,), jnp.int32),
            pltpu.VMEM((GROUP,), jnp.int32),
            pltpu.VMEM((GROUP, D), jnp.float32),
            pltpu.VMEM((GROUP, D), jnp.float32),
            pltpu.VMEM((GROUP, D), jnp.float32),
            pltpu.VMEM((GROUP, D), jnp.float32),
            pltpu.VMEM((GROUP,), jnp.int32),
            pltpu.VMEM((GROUP,), jnp.int32),
            pltpu.VMEM((GROUP,), jnp.int32),
            pltpu.VMEM((GROUP,), jnp.int32),
            pltpu.VMEM_SHARED((ACC_ROWS, D), jnp.float32),
            pltpu.SemaphoreType.DMA,
            pltpu.SemaphoreType.DMA,
            pltpu.SemaphoreType.DMA,
            pltpu.SemaphoreType.DMA,
            pltpu.SemaphoreType.DMA,
        ],
        compiler_params=pltpu.CompilerParams(needs_layout_passes=False),
    )
    return k(src, ptr_pad)


def kernel(src, indptr):
    ptr = indptr.astype(jnp.int32)
    ptr_pad = jnp.concatenate(
        [ptr, jnp.full((PTR_PAD - ptr.shape[0],), ptr[-1], jnp.int32)]
    )
    out = _run(src, ptr_pad)
    return out[:N_SEG]


# speedup vs baseline: 1.0731x; 1.0019x over previous
import jax
import jax.numpy as jnp
from jax import lax
from jax.experimental import pallas as pl
from jax.experimental.pallas import tpu as pltpu
from jax.experimental.pallas import tpu_sc as plsc

N_ROWS = 320000
N_SEG = 10000
D = 128
NC = 2
NS = 16
NW = NC * NS
SEG_PER_W = 320
SEG_PAD = NW * SEG_PER_W
PTR_SLICE = SEG_PER_W + 24
PTR_PAD = (NW - 1) * SEG_PER_W + PTR_SLICE
GROUP = 128
NBUF = 4
LANES = 16
KD = D // LANES
G = SEG_PER_W // LANES
ACC_ROWS = NS * SEG_PER_W + NS


def _pread(ref, i):
    return ref[pl.ds(i, LANES)][0]


def _sc_body(src_hbm, ptr_hbm, out_hbm, ptr_v, marks,
             buf0, buf1, buf2, buf3, ids0, ids1, ids2, ids3, acc,
             sem0, sem1, sem2, sem3, sem_sc):
    sid = lax.axis_index("s")
    cid = lax.axis_index("c")
    wid = sid * NC + cid
    seg0 = pl.multiple_of(wid * SEG_PER_W, 8)
    slot0 = pl.multiple_of(sid * SEG_PER_W, 8)
    dummy = NS * SEG_PER_W + sid

    bufs = (buf0, buf1, buf2, buf3)
    ids_refs = (ids0, ids1, ids2, ids3)
    sems = (sem0, sem1, sem2, sem3)

    pltpu.sync_copy(ptr_hbm.at[pl.ds(seg0, PTR_SLICE)], ptr_v)
    row_lo = _pread(ptr_v, 0)
    row_hi = _pread(ptr_v, SEG_PER_W)

    zf = jnp.zeros((LANES,), jnp.float32)
    zi = jnp.zeros((LANES,), jnp.int32)
    ones = jnp.ones((LANES,), jnp.int32)
    iota = lax.iota(jnp.int32, LANES)

    def zero_body(s, _):
        for k in range(KD):
            buf0[s, pl.ds(k * LANES, LANES)] = zf
        return 0

    lax.fori_loop(0, GROUP, zero_body, 0)
    for p, m in ((0, 128), (128, 128), (256, 64)):
        pltpu.sync_copy(buf0.at[pl.ds(0, m)], acc.at[pl.ds(slot0 + p, m)])

    row_lo_a = (row_lo // 8) * 8
    ngrp = (row_hi - row_lo_a + GROUP - 1) // GROUP

    def grp_off(t):
        off = row_lo_a + t * GROUP
        return off, pl.multiple_of(jnp.minimum(off, N_ROWS - GROUP), 8)

    def start_load(t, buf, sem):
        _, off_c = grp_off(t)
        pltpu.async_copy(src_hbm.at[pl.ds(off_c, GROUP)], buf, sem)

    @pl.when(0 < ngrp)
    def _():
        start_load(0, buf0, sem0)

    @pl.when(1 < ngrp)
    def _():
        start_load(1, buf1, sem1)

    def ring_body(g, base):
        for k in range(NBUF):
            t = g * NBUF + k
            kk = (k + 2) % NBUF

            @pl.when((t >= 2) & (t - 2 < ngrp))
            def _(kk=kk):
                pltpu.make_async_copy(
                    bufs[kk], acc.at[ids_refs[kk]], sem_sc
                ).wait()

            @pl.when(t + 2 < ngrp)
            def _(kk=kk, t=t):
                start_load(t + 2, bufs[kk], sems[kk])

            def fire(bs, t=t, k=k):
                off, off_c = grp_off(t)
                for j in range(GROUP // LANES):
                    marks[pl.ds(j * LANES, LANES)] = zi
                hi = off_c + GROUP

                def sm(q, _):
                    starts = ptr_v[pl.ds(q * LANES, LANES)]
                    m = (starts >= off) & (starts < hi)
                    plsc.addupdate_scatter(marks, [starts - off_c], ones, mask=m)
                    return 0

                lax.fori_loop(0, G, sm, 0)

                for j in range(GROUP // LANES):
                    mk = marks[pl.ds(j * LANES, LANES)]
                    csum = plsc.cumsum(mk)
                    idx16 = off_c + j * LANES + iota
                    valid = (idx16 >= off) & (idx16 >= row_lo) & (idx16 < row_hi)
                    slot = jnp.where(valid, slot0 + bs + csum - 1, dummy)
                    ids_refs[k][pl.ds(j * LANES, LANES)] = slot
                    bs = bs + csum[15]

                pltpu.make_async_copy(
                    src_hbm.at[pl.ds(off_c, GROUP)], bufs[k], sems[k]
                ).wait()
                pltpu.async_copy(
                    bufs[k], acc.at[ids_refs[k]], sem_sc, add=True
                )
                return bs

            base = lax.cond(t < ngrp, fire, lambda bs: bs, base)
        return base

    lax.fori_loop(0, (ngrp + 2 + NBUF - 1) // NBUF, ring_body, 0)

    for p, m in ((0, 128), (128, 128), (256, 64)):
        pltpu.sync_copy(acc.at[pl.ds(slot0 + p, m)], buf0.at[pl.ds(0, m)])

        def div_body(g2, _, p=p):
            cur16 = ptr_v[pl.ds(p + g2 * LANES, LANES)]
            nxt16 = plsc.load_gather(ptr_v, [p + g2 * LANES + 1 + iota])
            cntf = (nxt16 - cur16).astype(jnp.float32)
            recip = 1.0 / jnp.maximum(cntf, 1.0)
            for jj in range(LANES):
                rv = jnp.full((LANES,), recip[jj], jnp.float32)
                for k in range(KD):
                    sl = pl.ds(k * LANES, LANES)
                    buf0[g2 * LANES + jj, sl] = buf0[g2 * LANES + jj, sl] * rv
            return 0

        lax.fori_loop(0, m // LANES, div_body, 0)
        pltpu.sync_copy(buf0.at[pl.ds(0, m)], out_hbm.at[pl.ds(seg0 + p, m)])


@jax.jit
def _run(src, ptr_pad):
    mesh = plsc.VectorSubcoreMesh(core_axis_name="c", subcore_axis_name="s")
    k = pl.kernel(
        _sc_body,
        out_type=jax.ShapeDtypeStruct((SEG_PAD, D), jnp.float32),
        mesh=mesh,
        scratch_types=[
            pltpu.VMEM((PTR_SLICE,), jnp.int32),
            pltpu.VMEM((GROUP,), jnp.int32),
            pltpu.VMEM((GROUP, D), jnp.float32),
            pltpu.VMEM((GROUP, D), jnp.float32),
            pltpu.VMEM((GROUP, D), jnp.float32),
            pltpu.VMEM((GROUP, D), jnp.float32),
            pltpu.VMEM((GROUP,), jnp.int32),
            pltpu.VMEM((GROUP,), jnp.int32),
            pltpu.VMEM((GROUP,), jnp.int32),
            pltpu.VMEM((GROUP,), jnp.int32),
            pltpu.VMEM_SHARED((ACC_ROWS, D), jnp.float32),
            pltpu.SemaphoreType.DMA,
            pltpu.SemaphoreType.DMA,
            pltpu.SemaphoreType.DMA,
            pltpu.SemaphoreType.DMA,
            pltpu.SemaphoreType.DMA,
        ],
        compiler_params=pltpu.CompilerParams(needs_layout_passes=False),
    )
    return k(src, ptr_pad)


def kernel(src, indptr):
    ptr = indptr.astype(jnp.int32)
    ptr_pad = jnp.concatenate(
        [ptr, jnp.full((PTR_PAD - ptr.shape[0],), ptr[-1], jnp.int32)]
    )
    out = _run(src, ptr_pad)
    return out[:N_SEG]
